# baseline (speedup 1.0000x reference)
import functools

import jax
import jax.numpy as jnp
from jax import lax
from jax.experimental import pallas as pl
from jax.experimental.pallas import tpu as pltpu
from jax.experimental.pallas import tpu_sc as plsc

N_CORES = 2
N_SUBCORES = 16
NW = N_CORES * N_SUBCORES
LANES = 16
CHUNK = 128


def kernel(users, items, user_table, item_table):
    batch = users.shape[0]
    vocab, dim = user_table.shape
    lane_blocks = (vocab + 127) // 128
    b_per_w = batch // NW
    n_chunks = b_per_w // CHUNK

    ut = user_table.T
    it = item_table.T
    users2d = users.astype(jnp.int32).reshape(batch // CHUNK, CHUNK)
    items2d = items.astype(jnp.int32).reshape(batch // CHUNK, CHUNK)

    mesh = plsc.VectorSubcoreMesh(core_axis_name="c", subcore_axis_name="s")

    @functools.partial(
        pl.kernel,
        mesh=mesh,
        out_type=[
            jax.ShapeDtypeStruct((dim, batch), jnp.float32),
            jax.ShapeDtypeStruct((dim, batch), jnp.float32),
            jax.ShapeDtypeStruct((batch,), jnp.float32),
        ],
        scratch_types=[
            pltpu.VMEM((n_chunks, CHUNK), jnp.int32),
            pltpu.VMEM((n_chunks, CHUNK), jnp.int32),
            pltpu.VMEM((dim, b_per_w), jnp.float32),
            pltpu.VMEM((dim, b_per_w), jnp.float32),
            pltpu.VMEM((b_per_w,), jnp.float32),
            pltpu.SemaphoreType.DMA,
        ],
        compiler_params=pltpu.CompilerParams(use_tc_tiling_on_sc=False),
    )
    def gather_kernel(u_hbm, i_hbm, uidx_hbm, iidx_hbm, uo, io, so,
                      uix, iix, urows, irows, sc, sem):
        w = lax.axis_index("s") * N_CORES + lax.axis_index("c")
        base = w * b_per_w
        row0 = w * n_chunks
        pltpu.sync_copy(uidx_hbm.at[pl.ds(row0, n_chunks)], uix)
        pltpu.sync_copy(iidx_hbm.at[pl.ds(row0, n_chunks)], iix)

        copies = []
        for d in range(dim):
            for c in range(n_chunks):
                copies.append(pltpu.async_copy(
                    u_hbm.at[d].at[uix.at[c]],
                    urows.at[d, pl.ds(c * CHUNK, CHUNK)], sem))
                copies.append(pltpu.async_copy(
                    i_hbm.at[d].at[iix.at[c]],
                    irows.at[d, pl.ds(c * CHUNK, CHUNK)], sem))
        for cp in copies:
            cp.wait()

        @pl.loop(0, b_per_w, step=LANES)
        def _(j):
            acc = urows[0, pl.ds(j, LANES)] * irows[0, pl.ds(j, LANES)]
            for d in range(1, dim):
                acc = acc + urows[d, pl.ds(j, LANES)] * irows[d, pl.ds(j, LANES)]
            sc[pl.ds(j, LANES)] = acc

        pltpu.sync_copy(urows, uo.at[:, pl.ds(base, b_per_w)])
        pltpu.sync_copy(irows, io.at[:, pl.ds(base, b_per_w)])
        pltpu.sync_copy(sc, so.at[pl.ds(base, b_per_w)])

    uo_t, io_t, scores = gather_kernel(ut, it, users2d, items2d)
    return uo_t.T, io_t.T, scores


# speedup vs baseline: 4.2650x; 4.2650x over previous
import functools

import jax
import jax.numpy as jnp
from jax import lax
from jax.experimental import pallas as pl
from jax.experimental.pallas import tpu as pltpu
from jax.experimental.pallas import tpu_sc as plsc

N_CORES = 2
N_SUBCORES = 16
NW = N_CORES * N_SUBCORES
CHUNK = 128
TBLK = 8192


def _transpose_body(t_ref, o_ref):
    o_ref[...] = t_ref[...].T


def _relayout(table_t, vocab, dim):
    grid = (vocab + TBLK - 1) // TBLK
    return pl.pallas_call(
        _transpose_body,
        grid=(grid,),
        in_specs=[pl.BlockSpec((dim, TBLK), lambda i: (0, i))],
        out_specs=pl.BlockSpec((TBLK, dim), lambda i: (i, 0)),
        out_shape=jax.ShapeDtypeStruct((vocab, dim), jnp.float32),
    )(table_t)


def _score_body(u_ref, i_ref, o_ref):
    o_ref[...] = jnp.sum(u_ref[...] * i_ref[...], axis=1, keepdims=True)


def _sc_gather(user_table, item_table, users2d, items2d, batch, dim):
    b_per_w = batch // NW
    n_chunks = b_per_w // CHUNK
    mesh = plsc.VectorSubcoreMesh(core_axis_name="c", subcore_axis_name="s")

    @functools.partial(
        pl.kernel,
        mesh=mesh,
        out_type=[
            jax.ShapeDtypeStruct((batch, dim), jnp.float32),
            jax.ShapeDtypeStruct((batch, dim), jnp.float32),
        ],
        scratch_types=[
            pltpu.VMEM((n_chunks, CHUNK), jnp.int32),
            pltpu.VMEM((n_chunks, CHUNK), jnp.int32),
            pltpu.VMEM((b_per_w, dim), jnp.float32),
            pltpu.VMEM((b_per_w, dim), jnp.float32),
            pltpu.SemaphoreType.DMA,
        ],
        compiler_params=pltpu.CompilerParams(use_tc_tiling_on_sc=False),
    )
    def gather_kernel(u_tab, i_tab, u_idx_hbm, i_idx_hbm, u_out, i_out,
                      u_idx, i_idx, u_rows, i_rows, sem):
        wid = lax.axis_index("s") * N_CORES + lax.axis_index("c")
        base = wid * b_per_w
        row0 = wid * n_chunks
        pltpu.sync_copy(u_idx_hbm.at[pl.ds(row0, n_chunks)], u_idx)
        pltpu.sync_copy(i_idx_hbm.at[pl.ds(row0, n_chunks)], i_idx)
        copies = []
        for c in range(n_chunks):
            copies.append(pltpu.async_copy(
                u_tab.at[u_idx.at[c]], u_rows.at[pl.ds(c * CHUNK, CHUNK)], sem))
            copies.append(pltpu.async_copy(
                i_tab.at[i_idx.at[c]], i_rows.at[pl.ds(c * CHUNK, CHUNK)], sem))
        for cp in copies:
            cp.wait()
        pltpu.sync_copy(u_rows, u_out.at[pl.ds(base, b_per_w)])
        pltpu.sync_copy(i_rows, i_out.at[pl.ds(base, b_per_w)])

    return gather_kernel(user_table, item_table, users2d, items2d)


def kernel(users, items, user_table, item_table):
    batch = users.shape[0]
    vocab, dim = user_table.shape
    users2d = users.astype(jnp.int32).reshape(batch // CHUNK, CHUNK)
    items2d = items.astype(jnp.int32).reshape(batch // CHUNK, CHUNK)
    ut_rm = _relayout(user_table.T, vocab, dim)
    it_rm = _relayout(item_table.T, vocab, dim)
    user_emb, item_emb = _sc_gather(
        ut_rm, it_rm, users2d, items2d, batch, dim)
    scores2d = pl.pallas_call(
        _score_body,
        out_shape=jax.ShapeDtypeStruct((batch, 1), jnp.float32),
    )(user_emb, item_emb)
    return user_emb, item_emb, scores2d.reshape(batch)
